# Initial kernel scaffold; baseline (speedup 1.0000x reference)
import functools

import jax
import jax.numpy as jnp
from jax import lax
from jax.experimental import pallas as pl
from jax.experimental.pallas import tpu as pltpu
from jax.experimental.pallas import tpu_sc as plsc

BATCH = 4096
HIST = 50
VOCAB = 16384
EMB_DIM = 64
OUT_DIM = EMB_DIM + 1
ID_MASK = (1 << 14) - 1
BAD_MASK = (1 << 14) | (1 << 15)

NC = 2
NS = 16
NW = NC * NS
L = 16

B_TOTAL = BATCH * HIST
PER_W = B_TOTAL // NW
CHUNK = 1280
NCH = PER_W // CHUNK
GROWS = 128
NG = CHUNK // GROWS
VPG = GROWS // L


def _sc_body(qc_hbm, emb_hbm, out_hbm, qc_v, idx_v, buf_v, sem):
    wid = lax.axis_index("s") * NC + lax.axis_index("c")
    base = wid * PER_W

    def chunk_body(c, carry):
        cbase = base + c * CHUNK
        pltpu.sync_copy(qc_hbm.at[pl.ds(cbase, CHUNK)], qc_v)

        def idx_body(i, carry):
            j = i // VPG
            t = i % VPG
            q = qc_v[pl.ds(j * GROWS + t * L, L)]
            idx_v[j, pl.ds(t * L, L)] = q & ID_MASK
            return carry

        lax.fori_loop(0, NG * VPG, idx_body, 0)

        handles = [
            pltpu.async_copy(
                emb_hbm.at[idx_v.at[j]],
                buf_v.at[pl.ds(j * GROWS, GROWS)],
                sem,
            )
            for j in range(NG)
        ]
        for h in handles:
            h.wait()

        lane = lax.iota(jnp.int32, L)
        col = jnp.full((L,), EMB_DIM, jnp.int32)

        def valid_body(i, carry):
            q = qc_v[pl.ds(i * L, L)]
            v = jnp.where((q & BAD_MASK) == 0, 1.0, 0.0).astype(jnp.float32)
            plsc.store_scatter(buf_v, [lane + i * L, col], v)
            return carry

        lax.fori_loop(0, CHUNK // L, valid_body, 0)

        pltpu.sync_copy(buf_v, out_hbm.at[pl.ds(cbase, CHUNK)])
        return carry

    lax.fori_loop(0, NCH, chunk_body, 0)


_call = functools.partial(
    pl.kernel,
    out_type=jax.ShapeDtypeStruct((B_TOTAL, OUT_DIM), jnp.float32),
    mesh=plsc.VectorSubcoreMesh(core_axis_name="c", subcore_axis_name="s"),
    scratch_types=[
        pltpu.VMEM((CHUNK,), jnp.int32),
        pltpu.VMEM((NG, GROWS), jnp.int32),
        pltpu.VMEM((CHUNK, OUT_DIM), jnp.float32),
        pltpu.SemaphoreType.DMA,
    ],
)(_sc_body)


@jax.jit
def kernel(qc_flags, emb):
    qc_flat = qc_flags.astype(jnp.int32).reshape(B_TOTAL)
    emb_pad = jnp.pad(emb, ((0, 0), (0, 1)))
    out = _call(qc_flat, emb_pad)
    return out.reshape(BATCH, HIST, OUT_DIM)


# speedup vs baseline: 4.8085x; 4.8085x over previous
import functools

import jax
import jax.numpy as jnp
from jax import lax
from jax.experimental import pallas as pl
from jax.experimental.pallas import tpu as pltpu
from jax.experimental.pallas import tpu_sc as plsc

BATCH = 4096
HIST = 50
VOCAB = 16384
EMB_DIM = 64
OUT_DIM = EMB_DIM + 1
ID_MASK = (1 << 14) - 1
BAD_MASK = (1 << 14) | (1 << 15)

NC = 2
NS = 16
NW = NC * NS
L = 16

B_TOTAL = BATCH * HIST
PER_W = B_TOTAL // NW
CHUNK = 1280
NCH = PER_W // CHUNK
GROWS = 128
NG = CHUNK // GROWS
VPG = GROWS // L


def _sc_body(qc_hbm, emb_hbm, feat_hbm, val_hbm, qc_v, idx_v, val_v, buf_v, sem):
    wid = lax.axis_index("s") * NC + lax.axis_index("c")
    base = wid * PER_W

    def chunk_body(c, carry):
        cbase = base + c * CHUNK
        pltpu.sync_copy(qc_hbm.at[pl.ds(cbase, CHUNK)], qc_v)

        def idx_body(i, carry):
            j = i // VPG
            t = i % VPG
            q = qc_v[pl.ds(j * GROWS + t * L, L)]
            idx_v[j, pl.ds(t * L, L)] = q & ID_MASK
            val_v[pl.ds(j * GROWS + t * L, L)] = jnp.where(
                (q & BAD_MASK) == 0, 1.0, 0.0
            )
            return carry

        lax.fori_loop(0, NG * VPG, idx_body, 0)

        handles = [
            pltpu.async_copy(
                emb_hbm.at[idx_v.at[j]],
                buf_v.at[pl.ds(j * GROWS, GROWS)],
                sem,
            )
            for j in range(NG)
        ]
        for h in handles:
            h.wait()

        pltpu.sync_copy(buf_v, feat_hbm.at[pl.ds(cbase, CHUNK)])
        pltpu.sync_copy(val_v, val_hbm.at[pl.ds(cbase, CHUNK)])
        return carry

    lax.fori_loop(0, NCH, chunk_body, 0)


_call = functools.partial(
    pl.kernel,
    out_type=(
        jax.ShapeDtypeStruct((B_TOTAL, EMB_DIM), jnp.float32),
        jax.ShapeDtypeStruct((B_TOTAL,), jnp.float32),
    ),
    mesh=plsc.VectorSubcoreMesh(core_axis_name="c", subcore_axis_name="s"),
    scratch_types=[
        pltpu.VMEM((CHUNK,), jnp.int32),
        pltpu.VMEM((NG, GROWS), jnp.int32),
        pltpu.VMEM((CHUNK,), jnp.float32),
        pltpu.VMEM((CHUNK, EMB_DIM), jnp.float32),
        pltpu.SemaphoreType.DMA,
    ],
    compiler_params=pltpu.CompilerParams(use_tc_tiling_on_sc=False),
)(_sc_body)


@jax.jit
def kernel(qc_flags, emb):
    qc_flat = qc_flags.astype(jnp.int32).reshape(B_TOTAL)
    feat, valid = _call(qc_flat, emb)
    return jnp.concatenate(
        [
            feat.reshape(BATCH, HIST, EMB_DIM),
            valid.reshape(BATCH, HIST, 1),
        ],
        axis=-1,
    )
